# baseline (speedup 1.0000x reference)
import functools

import jax
import jax.numpy as jnp
from jax import lax
from jax.experimental import pallas as pl
from jax.experimental.pallas import tpu as pltpu
from jax.experimental.pallas import tpu_sc as plsc

_DIM = 4096
_N_EXPERTS = 64
_TOPK = 8
_N_GROUPS = 8
_TOPK_GROUPS = 4
_ROUTE_SCALE = 2.5
_NEG = float("-inf")

_NW = 32
_TPW = 512
_L = 16


def _tc_scores_kernel(x_ref, w_ref, b_ref, sco_ref, s_ref):
    x = x_ref[...]
    w = w_ref[...]
    b = b_ref[...]
    logits = jax.lax.dot_general(
        w, x, (((1,), (1,)), ((), ())), preferred_element_type=jnp.float32)
    scores = jax.nn.sigmoid(logits + b)
    sco_ref[...] = scores[None]
    s_ref[...] = (scores + b)[None]


def _sc_route_body(sco_hbm, s_hbm, wout_hbm, iout_hbm, sco_v, s_v, wv, iv):
    wid = lax.axis_index("s") * 2 + lax.axis_index("c")
    pltpu.sync_copy(sco_hbm.at[wid], sco_v)
    pltpu.sync_copy(s_hbm.at[wid], s_v)

    def block(j, carry):
        base = j * _L
        svals = [s_v[e, pl.ds(base, _L)] for e in range(_N_EXPERTS)]

        pens = []
        gsums = []
        for g in range(_N_GROUPS):
            m1 = svals[g * 8]
            m2 = jnp.full((_L,), _NEG, jnp.float32)
            for e in range(1, 8):
                v = svals[g * 8 + e]
                m2 = jnp.maximum(m2, jnp.minimum(m1, v))
                m1 = jnp.maximum(m1, v)
            gsums.append(m1 + m2)
            pens.append(jnp.full((_L,), _NEG, jnp.float32))

        work = list(gsums)
        for _k in range(_TOPK_GROUPS):
            m = work[0]
            for g in range(1, _N_GROUPS):
                m = jnp.maximum(m, work[g])
            gi = jnp.full((_L,), _N_GROUPS, jnp.int32)
            for g in range(_N_GROUPS - 1, -1, -1):
                gi = jnp.where(work[g] == m, g, gi)
            for g in range(_N_GROUPS):
                hit = gi == g
                work[g] = jnp.where(hit, _NEG, work[g])
                pens[g] = jnp.where(hit, 0.0, pens[g])

        tm = [jnp.full((_L,), _NEG, jnp.float32) for _ in range(_TOPK)]
        ti = [jnp.full((_L,), 0, jnp.int32) for _ in range(_TOPK)]
        tw = [jnp.full((_L,), 0.0, jnp.float32) for _ in range(_TOPK)]
        for e in range(_N_EXPERTS):
            v = svals[e] + pens[e // 8]
            ic = jnp.full((_L,), e, jnp.int32)
            wc = sco_v[e, pl.ds(base, _L)]
            for lvl in range(_TOPK):
                c = v > tm[lvl]
                nm = jnp.where(c, v, tm[lvl])
                ni = jnp.where(c, ic, ti[lvl])
                nw = jnp.where(c, wc, tw[lvl])
                v = jnp.where(c, tm[lvl], v)
                ic = jnp.where(c, ti[lvl], ic)
                wc = jnp.where(c, tw[lvl], wc)
                tm[lvl], ti[lvl], tw[lvl] = nm, ni, nw

        tot = tw[0]
        for k in range(1, _TOPK):
            tot = tot + tw[k]
        scale = _ROUTE_SCALE / tot
        for k in range(_TOPK):
            wv[k, pl.ds(base, _L)] = tw[k] * scale
            iv[k, pl.ds(base, _L)] = ti[k]
        return carry

    lax.fori_loop(0, _TPW // _L, block, 0)
    pltpu.sync_copy(wv, wout_hbm.at[wid])
    pltpu.sync_copy(iv, iout_hbm.at[wid])


_sc_route = functools.partial(
    pl.kernel,
    out_type=[
        jax.ShapeDtypeStruct((_NW, _TOPK, _TPW), jnp.float32),
        jax.ShapeDtypeStruct((_NW, _TOPK, _TPW), jnp.int32),
    ],
    mesh=plsc.VectorSubcoreMesh(core_axis_name="c", subcore_axis_name="s"),
    scratch_types=[
        pltpu.VMEM((_N_EXPERTS, _TPW), jnp.float32),
        pltpu.VMEM((_N_EXPERTS, _TPW), jnp.float32),
        pltpu.VMEM((_TOPK, _TPW), jnp.float32),
        pltpu.VMEM((_TOPK, _TPW), jnp.int32),
    ],
)(_sc_route_body)


@jax.jit
def kernel(x, weight, bias):
    t = x.shape[0]
    tb = _TPW
    b2 = bias.reshape(_N_EXPERTS, 1)
    sco_b, s_b = pl.pallas_call(
        _tc_scores_kernel,
        grid=(t // tb,),
        in_specs=[
            pl.BlockSpec((tb, _DIM), lambda i: (i, 0)),
            pl.BlockSpec((_N_EXPERTS, _DIM), lambda i: (0, 0)),
            pl.BlockSpec((_N_EXPERTS, 1), lambda i: (0, 0)),
        ],
        out_specs=[
            pl.BlockSpec((1, _N_EXPERTS, tb), lambda i: (i, 0, 0)),
            pl.BlockSpec((1, _N_EXPERTS, tb), lambda i: (i, 0, 0)),
        ],
        out_shape=[
            jax.ShapeDtypeStruct((t // tb, _N_EXPERTS, tb), jnp.float32),
            jax.ShapeDtypeStruct((t // tb, _N_EXPERTS, tb), jnp.float32),
        ],
    )(x, weight, b2)
    wts_b, idx_b = _sc_route(sco_b, s_b)
    wts = wts_b.transpose(0, 2, 1).reshape(t, _TOPK)
    idx = idx_b.transpose(0, 2, 1).reshape(t, _TOPK)
    return wts, idx


# speedup vs baseline: 1.2776x; 1.2776x over previous
import functools

import jax
import jax.numpy as jnp
from jax.experimental import pallas as pl

_DIM = 4096
_N_EXPERTS = 64
_TOPK = 8
_N_GROUPS = 8
_GROUP_SIZE = _N_EXPERTS // _N_GROUPS
_TOPK_GROUPS = 4
_ROUTE_SCALE = 2.5

_NEG = float("-inf")


def _router_kernel(x_ref, w_ref, b_ref, wout_ref, iout_ref):
    x = x_ref[...]
    w = w_ref[...]
    b = b_ref[...]

    logits = jax.lax.dot_general(
        w, x, (((1,), (1,)), ((), ())), preferred_element_type=jnp.float32)
    scores = jax.nn.sigmoid(logits + b)
    s = scores + b

    tb = x.shape[0]

    row8 = jax.lax.broadcasted_iota(jnp.int32, (_GROUP_SIZE, tb), 0)
    gs_rows = []
    for g in range(_N_GROUPS):
        slab = s[g * _GROUP_SIZE:(g + 1) * _GROUP_SIZE, :]
        m1 = jnp.max(slab, axis=0, keepdims=True)
        r1 = jnp.min(jnp.where(slab == m1, row8, _GROUP_SIZE), axis=0,
                     keepdims=True)
        m2 = jnp.max(jnp.where(row8 == r1, _NEG, slab), axis=0,
                     keepdims=True)
        gs_rows.append(m1 + m2)
    gscores = jnp.concatenate(gs_rows, axis=0)

    grow = jax.lax.broadcasted_iota(jnp.int32, (_N_GROUPS, tb), 0)
    sel = jnp.zeros((_N_GROUPS, tb), dtype=jnp.bool_)
    gtmp = gscores
    for _ in range(_TOPK_GROUPS):
        gm = jnp.max(gtmp, axis=0, keepdims=True)
        gl = jnp.min(jnp.where(gtmp == gm, grow, _N_GROUPS), axis=0,
                     keepdims=True)
        sel = sel | (grow == gl)
        gtmp = jnp.where(grow == gl, _NEG, gtmp)

    sm_rows = []
    for g in range(_N_GROUPS):
        slab = s[g * _GROUP_SIZE:(g + 1) * _GROUP_SIZE, :]
        sm_rows.append(jnp.where(sel[g:g + 1, :], slab, _NEG))
    sm = jnp.concatenate(sm_rows, axis=0)

    row64 = jax.lax.broadcasted_iota(jnp.int32, (_N_EXPERTS, tb), 0)
    idx_rows, w_rows = [], []
    for _ in range(_TOPK):
        m = jnp.max(sm, axis=0, keepdims=True)
        l = jnp.min(jnp.where(sm == m, row64, _N_EXPERTS), axis=0,
                    keepdims=True)
        hit = row64 == l
        w_rows.append(jnp.max(jnp.where(hit, scores, _NEG), axis=0,
                              keepdims=True))
        idx_rows.append(l)
        sm = jnp.where(hit, _NEG, sm)

    idx = jnp.concatenate(idx_rows, axis=0)
    wts = jnp.concatenate(w_rows, axis=0)
    wts = wts * (_ROUTE_SCALE / jnp.sum(wts, axis=0, keepdims=True))

    wout_ref[...] = wts.T
    iout_ref[...] = idx.T


@functools.partial(jax.jit, static_argnames=())
def kernel(x, weight, bias):
    t = x.shape[0]
    tb = 1024
    b2 = bias.reshape(_N_EXPERTS, 1)
    wts, idx = pl.pallas_call(
        _router_kernel,
        grid=(t // tb,),
        in_specs=[
            pl.BlockSpec((tb, _DIM), lambda i: (i, 0)),
            pl.BlockSpec((_N_EXPERTS, _DIM), lambda i: (0, 0)),
            pl.BlockSpec((_N_EXPERTS, 1), lambda i: (0, 0)),
        ],
        out_specs=[
            pl.BlockSpec((tb, _TOPK), lambda i: (i, 0)),
            pl.BlockSpec((tb, _TOPK), lambda i: (i, 0)),
        ],
        out_shape=[
            jax.ShapeDtypeStruct((t, _TOPK), jnp.float32),
            jax.ShapeDtypeStruct((t, _TOPK), jnp.int32),
        ],
    )(x, weight, b2)
    return wts, idx
